# baseline (speedup 1.0000x reference)
import functools

import jax
import jax.numpy as jnp
from jax import lax
from jax.experimental import pallas as pl
from jax.experimental.pallas import tpu as pltpu
from jax.experimental.pallas import tpu_sc as plsc

_NUM_CORES = 2
_NUM_SUBCORES = 16
_NUM_WORKERS = _NUM_CORES * _NUM_SUBCORES
_LANES = 16
_SUB = 8
_VT = 4096


def _repack_body(t_ref, *o_refs):
    x = t_ref[...]
    for s, o_ref in enumerate(o_refs):
        o_ref[...] = x[s]


def _repack_table(table_t, pitch):
    d, v = table_t.shape
    npj = pitch // _VT
    plane = jax.ShapeDtypeStruct(((d // _SUB) * pitch,), jnp.float32)
    return pl.pallas_call(
        _repack_body,
        grid=(d // _SUB, npj),
        in_specs=[pl.BlockSpec((_SUB, _VT), lambda a, j: (a, j))],
        out_specs=[
            pl.BlockSpec((_VT,), lambda a, j, npj=npj: (a * npj + j,))
            for _ in range(_SUB)
        ],
        out_shape=[plane] * _SUB,
        compiler_params=pltpu.CompilerParams(
            dimension_semantics=("parallel", "parallel"),
        ),
    )(table_t)


def _sc_gather_t(idx, planes, n_dims, pitch):
    (n,) = idx.shape
    n_tok_grp = _NUM_WORKERS // (n_dims // _SUB)
    per_w = n // n_tok_grp
    mesh = plsc.VectorSubcoreMesh(core_axis_name="c", subcore_axis_name="s")

    @functools.partial(
        pl.kernel,
        mesh=mesh,
        out_type=jax.ShapeDtypeStruct((n_dims * n,), jnp.float32),
        compiler_params=pltpu.CompilerParams(use_tc_tiling_on_sc=False),
        scratch_types=[
            pltpu.VMEM((per_w,), jnp.int32),
            pltpu.VMEM((_SUB * per_w,), jnp.float32),
            pltpu.SemaphoreType.DMA,
        ],
    )
    def gather_kernel(idx_hbm, *rest):
        tabs = rest[:_SUB]
        out_hbm = rest[_SUB]
        idx_v, xt_v, sem = rest[_SUB + 1:]
        wid = lax.axis_index("s") * _NUM_CORES + lax.axis_index("c")
        g = wid // n_tok_grp
        q = wid % n_tok_grp
        base = q * per_w
        pltpu.sync_copy(idx_hbm.at[pl.ds(base, per_w)], idx_v)
        for k in range(per_w // _LANES):
            sl = pl.ds(k * _LANES, _LANES)
            idx_v[sl] = idx_v[sl] + g * pitch
        copies = [
            pltpu.async_copy(
                tabs[s].at[idx_v],
                xt_v.at[pl.ds(s * per_w, per_w)],
                sem,
            )
            for s in range(_SUB)
        ]
        for c in copies:
            c.wait()
        for s in range(_SUB):
            pltpu.sync_copy(
                xt_v.at[pl.ds(s * per_w, per_w)],
                out_hbm.at[pl.ds((g * _SUB + s) * n + base, per_w)],
            )

    return gather_kernel(idx, *planes)


def _matmul_body(xt_ref, post_ref, w_ref, b_ref, out_ref, xpt_ref):
    @pl.when(pl.program_id(0) == 0)
    def _():
        xpt_ref[...] = xt_ref[...] + post_ref[...]

    acc = jax.lax.dot_general(
        w_ref[...],
        xpt_ref[...],
        (((0,), (0,)), ((), ())),
        preferred_element_type=jnp.float32,
    )
    out_ref[...] = acc + b_ref[...].T


def _lm_head(xt, post, w, b2, v_tile):
    d, t = post.shape
    v = w.shape[1]
    nvt = pl.cdiv(v, v_tile)
    return pl.pallas_call(
        _matmul_body,
        grid=(nvt,),
        in_specs=[
            pl.BlockSpec((d, t), lambda j: (0, 0)),
            pl.BlockSpec((d, t), lambda j: (0, 0)),
            pl.BlockSpec((d, v_tile), lambda j: (0, j)),
            pl.BlockSpec((1, v_tile), lambda j: (0, j)),
        ],
        out_specs=pl.BlockSpec((v_tile, t), lambda j: (j, 0)),
        out_shape=jax.ShapeDtypeStruct((v, t), jnp.float32),
        scratch_shapes=[pltpu.VMEM((d, t), jnp.float32)],
        compiler_params=pltpu.CompilerParams(
            dimension_semantics=("arbitrary",),
            fuse_transposed_lhs_in_matmul=True,
        ),
    )(xt, post, w, b2)


def kernel(indices, token_table, pos_table, W, b):
    batch, seq = indices.shape
    vocab, dim = token_table.shape
    idx = indices.reshape(-1).astype(jnp.int32)
    pitch = pl.cdiv(vocab, _VT) * _VT
    planes = _repack_table(token_table.T, pitch)
    xt = _sc_gather_t(idx, planes, dim, pitch).reshape(dim, batch * seq)
    logits_t = _lm_head(
        xt, pos_table[:seq].T, W, b.reshape(1, -1), v_tile=2048
    )
    return logits_t.T[None]


# speedup vs baseline: 1.2355x; 1.2355x over previous
import functools

import jax
import jax.numpy as jnp
from jax import lax
from jax.experimental import pallas as pl
from jax.experimental.pallas import tpu as pltpu
from jax.experimental.pallas import tpu_sc as plsc

_NUM_CORES = 2
_NUM_SUBCORES = 16
_NUM_WORKERS = _NUM_CORES * _NUM_SUBCORES
_LANES = 16
_FIRE = 16


def _sc_gather_t(idx, flat_t, n_dims, vocab):
    (n,) = idx.shape
    d_grp = 8
    n_tok_grp = _NUM_WORKERS // (n_dims // d_grp)
    per_w = n // n_tok_grp
    mesh = plsc.VectorSubcoreMesh(core_axis_name="c", subcore_axis_name="s")

    @functools.partial(
        pl.kernel,
        mesh=mesh,
        out_type=jax.ShapeDtypeStruct((n_dims * n,), jnp.float32),
        compiler_params=pltpu.CompilerParams(use_tc_tiling_on_sc=False),
        scratch_types=[
            pltpu.VMEM((per_w,), jnp.int32),
            pltpu.VMEM((d_grp * per_w,), jnp.int32),
            pltpu.VMEM((d_grp * per_w,), jnp.float32),
            pltpu.SemaphoreType.DMA,
        ],
    )
    def gather_kernel(idx_hbm, tab_hbm, out_hbm, idx_v, idxs_v, xt_v, sem):
        wid = lax.axis_index("s") * _NUM_CORES + lax.axis_index("c")
        g = wid // n_tok_grp
        q = wid % n_tok_grp
        base = q * per_w
        pltpu.sync_copy(idx_hbm.at[pl.ds(base, per_w)], idx_v)
        for k in range(per_w // _LANES):
            sl = pl.ds(k * _LANES, _LANES)
            v = idx_v[sl] + g * (d_grp * vocab)
            for d in range(d_grp):
                idxs_v[pl.ds(d * per_w + k * _LANES, _LANES)] = v + d * vocab
        copies = [
            pltpu.async_copy(
                tab_hbm.at[idxs_v.at[pl.ds(d * per_w, per_w)]],
                xt_v.at[pl.ds(d * per_w, per_w)],
                sem,
            )
            for d in range(d_grp)
        ]
        for c in copies:
            c.wait()
        for d in range(d_grp):
            pltpu.sync_copy(
                xt_v.at[pl.ds(d * per_w, per_w)],
                out_hbm.at[pl.ds((g * d_grp + d) * n + base, per_w)],
            )

    return gather_kernel(idx, flat_t)


def _matmul_body(xt_ref, post_ref, w_ref, b_ref, out_ref, xpt_ref):
    @pl.when(pl.program_id(0) == 0)
    def _():
        xpt_ref[...] = xt_ref[...] + post_ref[...]

    acc = jax.lax.dot_general(
        w_ref[...],
        xpt_ref[...],
        (((0,), (0,)), ((), ())),
        preferred_element_type=jnp.float32,
    )
    out_ref[...] = acc + b_ref[...].T


def _lm_head(xt, post, w, b2, v_tile):
    d, t = post.shape
    v = w.shape[1]
    nvt = pl.cdiv(v, v_tile)
    return pl.pallas_call(
        _matmul_body,
        grid=(nvt,),
        in_specs=[
            pl.BlockSpec((d, t), lambda j: (0, 0)),
            pl.BlockSpec((d, t), lambda j: (0, 0)),
            pl.BlockSpec((d, v_tile), lambda j: (0, j)),
            pl.BlockSpec((1, v_tile), lambda j: (0, j)),
        ],
        out_specs=pl.BlockSpec((v_tile, t), lambda j: (j, 0)),
        out_shape=jax.ShapeDtypeStruct((v, t), jnp.float32),
        scratch_shapes=[pltpu.VMEM((d, t), jnp.float32)],
        compiler_params=pltpu.CompilerParams(
            dimension_semantics=("arbitrary",),
            fuse_transposed_lhs_in_matmul=True,
        ),
    )(xt, post, w, b2)


def kernel(indices, token_table, pos_table, W, b):
    batch, seq = indices.shape
    vocab, dim = token_table.shape
    idx = indices.reshape(-1).astype(jnp.int32)
    pitch = ((vocab + 127) // 128) * 128
    flat_t = jnp.pad(token_table.T, ((0, 0), (0, pitch - vocab))).reshape(-1)
    xt = _sc_gather_t(idx, flat_t, dim, pitch).reshape(dim, batch * seq)
    logits_t = _lm_head(
        xt, pos_table[:seq].T, W, b.reshape(1, -1), v_tile=2048
    )
    return logits_t.T[None]
